# baseline (speedup 1.0000x reference)
import functools
import math

import jax
import jax.numpy as jnp
import numpy as np
from jax import lax
from jax.experimental import pallas as pl
from jax.experimental.pallas import tpu as pltpu
from jax.experimental.pallas import tpu_sc as plsc

_DIM = 1024
_HALF = _DIM // 2
_BASE = 10000.0
_DIST_SCALE = 16.0
_INV_SQRT_DIM = 1.0 / math.sqrt(_DIM)


def _fit_turn_polys():
    v = np.linspace(-0.5, 0.5, 40001)
    w = v * v
    cosy = np.cos(2 * np.pi * v)
    siny = np.where(v == 0, 2 * np.pi, np.sin(2 * np.pi * v) / np.where(v == 0, 1, v))
    pc = np.polynomial.Polynomial.fit(w, cosy, 7).convert().coef
    ps = np.polynomial.Polynomial.fit(w, siny, 7).convert().coef
    return (tuple(float(c) * _INV_SQRT_DIM for c in pc),
            tuple(float(c) * _INV_SQRT_DIM for c in ps))


_COS_COEF, _SIN_COEF = _fit_turn_polys()


def _freq_reduction_consts():
    inv_freq32 = (
        1.0 / (_BASE ** (np.arange(0, _DIM, 2).astype(np.float32) / np.float32(_DIM)))
    ).astype(np.float32)
    f = np.mod(_DIST_SCALE * inv_freq32.astype(np.float64) / (2 * np.pi), 1.0)
    f_hi = (np.round(f * 4096.0) / 4096.0).astype(np.float32)
    f_lo = (f - f_hi.astype(np.float64)).astype(np.float32)
    return f_hi.reshape(1, _HALF), f_lo.reshape(1, _HALF)


_F_HI, _F_LO = _freq_reduction_consts()


def _sc_gather(weight, idx_flat, n_tokens):
    info = plsc.get_sparse_core_info()
    nw = info.num_cores * info.num_subcores
    b_per_w = n_tokens // nw
    chunk = 64
    n_chunks = b_per_w // chunk
    mesh = plsc.VectorSubcoreMesh(core_axis_name="c", subcore_axis_name="s")

    @functools.partial(
        pl.kernel,
        mesh=mesh,
        out_type=jax.ShapeDtypeStruct((n_tokens, _DIM), jnp.float32),
        scratch_types=[
            pltpu.VMEM((b_per_w,), jnp.int32),
            pltpu.VMEM((chunk, _DIM), jnp.float32),
            pltpu.SemaphoreType.DMA,
        ],
    )
    def gather_kernel(table_hbm, idx_hbm, out_hbm, idx_v, rows_v, sem):
        wid = lax.axis_index("s") * info.num_cores + lax.axis_index("c")
        base = wid * b_per_w
        pltpu.sync_copy(idx_hbm.at[pl.ds(base, b_per_w)], idx_v)

        def body(j, carry):
            pltpu.async_copy(
                table_hbm.at[idx_v.at[pl.ds(j * chunk, chunk)]], rows_v, sem
            ).wait()
            pltpu.sync_copy(rows_v, out_hbm.at[pl.ds(base + j * chunk, chunk)])
            return carry

        lax.fori_loop(0, n_chunks, body, 0)

    return gather_kernel(weight, idx_flat)


def _horner(w, coef):
    acc = jnp.full_like(w, coef[-1])
    for c in coef[-2::-1]:
        acc = acc * w + c
    return acc


def _rotary_body(f_hi_ref, f_lo_ref, n_ref, x_ref, o_ref):
    n = n_ref[...]
    f_hi = f_hi_ref[...]
    f_lo = f_lo_ref[...]
    t = n * f_hi
    u = t - jnp.floor(t) + n * f_lo
    v = u - jnp.floor(u + 0.5)
    w = v * v
    c = _horner(w, _COS_COEF)
    s = v * _horner(w, _SIN_COEF)
    x1 = x_ref[:, :_HALF]
    x2 = x_ref[:, _HALF:]
    o_ref[:, :_HALF] = x1 * c - x2 * s
    o_ref[:, _HALF:] = x2 * c + x1 * s


def _rotary_tc(embeds, nsub):
    n_tokens = embeds.shape[0]
    t = 256
    grid = (n_tokens // t,)
    return pl.pallas_call(
        _rotary_body,
        grid=grid,
        in_specs=[
            pl.BlockSpec((1, _HALF), lambda i: (0, 0)),
            pl.BlockSpec((1, _HALF), lambda i: (0, 0)),
            pl.BlockSpec((t, 1), lambda i: (i, 0)),
            pl.BlockSpec((t, _DIM), lambda i: (i, 0)),
        ],
        out_specs=pl.BlockSpec((t, _DIM), lambda i: (i, 0)),
        out_shape=jax.ShapeDtypeStruct((n_tokens, _DIM), jnp.float32),
    )(jnp.asarray(_F_HI), jnp.asarray(_F_LO), nsub, embeds)


def kernel(ids, ids_sub, weight):
    b, s = ids.shape
    n = b * s
    idx = ids.reshape(n)
    nsub = ids_sub.astype(jnp.float32).reshape(n, 1)
    n_chunks = 4
    cn = n // n_chunks
    outs = []
    for ci in range(n_chunks):
        embeds = _sc_gather(weight, lax.slice(idx, (ci * cn,), ((ci + 1) * cn,)), cn)
        outs.append(_rotary_tc(embeds, lax.slice(nsub, (ci * cn, 0), ((ci + 1) * cn, 1))))
    return jnp.concatenate(outs, axis=0).reshape(b, s, _DIM)


# speedup vs baseline: 1.2679x; 1.2679x over previous
import functools
import math

import jax
import jax.numpy as jnp
import numpy as np
from jax import lax
from jax.experimental import pallas as pl
from jax.experimental.pallas import tpu as pltpu
from jax.experimental.pallas import tpu_sc as plsc

_DIM = 1024
_HALF = _DIM // 2
_BASE = 10000.0
_DIST_SCALE = 16.0
_INV_SQRT_DIM = 1.0 / math.sqrt(_DIM)


def _fit_turn_polys():
    v = np.linspace(-0.5, 0.5, 40001)
    w = v * v
    cosy = np.cos(2 * np.pi * v)
    siny = np.where(v == 0, 2 * np.pi, np.sin(2 * np.pi * v) / np.where(v == 0, 1, v))
    pc = np.polynomial.Polynomial.fit(w, cosy, 5).convert().coef
    ps = np.polynomial.Polynomial.fit(w, siny, 5).convert().coef
    return (tuple(float(c) * _INV_SQRT_DIM for c in pc),
            tuple(float(c) * _INV_SQRT_DIM for c in ps))


_COS_COEF, _SIN_COEF = _fit_turn_polys()


def _freq_reduction_consts():
    inv_freq32 = (
        1.0 / (_BASE ** (np.arange(0, _DIM, 2).astype(np.float32) / np.float32(_DIM)))
    ).astype(np.float32)
    f = np.mod(_DIST_SCALE * inv_freq32.astype(np.float64) / (2 * np.pi), 1.0)
    f_hi = (np.round(f * 4096.0) / 4096.0).astype(np.float32)
    f_lo = (f - f_hi.astype(np.float64)).astype(np.float32)
    return f_hi.reshape(1, _HALF), f_lo.reshape(1, _HALF)


_F_HI, _F_LO = _freq_reduction_consts()


def _sc_gather(weight, idx_flat, n_tokens):
    info = plsc.get_sparse_core_info()
    nw = info.num_cores * info.num_subcores
    b_per_w = n_tokens // nw
    chunk = 64
    n_chunks = b_per_w // chunk
    mesh = plsc.VectorSubcoreMesh(core_axis_name="c", subcore_axis_name="s")

    @functools.partial(
        pl.kernel,
        mesh=mesh,
        out_type=jax.ShapeDtypeStruct((n_tokens, _DIM), jnp.float32),
        scratch_types=[
            pltpu.VMEM((b_per_w,), jnp.int32),
            pltpu.VMEM((chunk, _DIM), jnp.float32),
            pltpu.SemaphoreType.DMA,
        ],
    )
    def gather_kernel(table_hbm, idx_hbm, out_hbm, idx_v, rows_v, sem):
        wid = lax.axis_index("s") * info.num_cores + lax.axis_index("c")
        base = wid * b_per_w
        pltpu.sync_copy(idx_hbm.at[pl.ds(base, b_per_w)], idx_v)

        def body(j, carry):
            pltpu.async_copy(
                table_hbm.at[idx_v.at[pl.ds(j * chunk, chunk)]], rows_v, sem
            ).wait()
            pltpu.sync_copy(rows_v, out_hbm.at[pl.ds(base + j * chunk, chunk)])
            return carry

        lax.fori_loop(0, n_chunks, body, 0)

    return gather_kernel(weight, idx_flat)


def _horner(w, coef):
    acc = jnp.full_like(w, coef[-1])
    for c in coef[-2::-1]:
        acc = acc * w + c
    return acc


def _rotary_body(f_hi_ref, f_lo_ref, n_ref, x_ref, o_ref):
    n = n_ref[...]
    f_hi = f_hi_ref[...]
    f_lo = f_lo_ref[...]
    z = n * f_hi + n * f_lo
    v = z - jnp.floor(z + 0.5)
    w = v * v
    c = _horner(w, _COS_COEF)
    s = v * _horner(w, _SIN_COEF)
    x1 = x_ref[:, :_HALF]
    x2 = x_ref[:, _HALF:]
    o_ref[:, :_HALF] = x1 * c - x2 * s
    o_ref[:, _HALF:] = x2 * c + x1 * s


def _rotary_tc(embeds, nsub):
    n_tokens = embeds.shape[0]
    t = 256
    grid = (n_tokens // t,)
    return pl.pallas_call(
        _rotary_body,
        grid=grid,
        in_specs=[
            pl.BlockSpec((1, _HALF), lambda i: (0, 0)),
            pl.BlockSpec((1, _HALF), lambda i: (0, 0)),
            pl.BlockSpec((t, 1), lambda i: (i, 0)),
            pl.BlockSpec((t, _DIM), lambda i: (i, 0)),
        ],
        out_specs=pl.BlockSpec((t, _DIM), lambda i: (i, 0)),
        out_shape=jax.ShapeDtypeStruct((n_tokens, _DIM), jnp.float32),
    )(jnp.asarray(_F_HI), jnp.asarray(_F_LO), nsub, embeds)


def kernel(ids, ids_sub, weight):
    b, s = ids.shape
    n = b * s
    idx = ids.reshape(n)
    nsub = ids_sub.astype(jnp.float32).reshape(n, 1)
    embeds = _sc_gather(weight, idx, n)
    out = _rotary_tc(embeds, nsub)
    return out.reshape(b, s, _DIM)


# speedup vs baseline: 1.2991x; 1.0247x over previous
import functools
import math

import jax
import jax.numpy as jnp
import numpy as np
from jax import lax
from jax.experimental import pallas as pl
from jax.experimental.pallas import tpu as pltpu
from jax.experimental.pallas import tpu_sc as plsc

_DIM = 1024
_HALF = _DIM // 2
_BASE = 10000.0
_DIST_SCALE = 16.0
_INV_SQRT_DIM = 1.0 / math.sqrt(_DIM)


def _fit_turn_polys():
    v = np.linspace(-0.5, 0.5, 40001)
    w = v * v
    cosy = np.cos(2 * np.pi * v)
    siny = np.where(v == 0, 2 * np.pi, np.sin(2 * np.pi * v) / np.where(v == 0, 1, v))
    pc = np.polynomial.Polynomial.fit(w, cosy, 5).convert().coef
    ps = np.polynomial.Polynomial.fit(w, siny, 5).convert().coef
    return (tuple(float(c) * _INV_SQRT_DIM for c in pc),
            tuple(float(c) * _INV_SQRT_DIM for c in ps))


_COS_COEF, _SIN_COEF = _fit_turn_polys()


def _freq_reduction_consts():
    inv_freq32 = (
        1.0 / (_BASE ** (np.arange(0, _DIM, 2).astype(np.float32) / np.float32(_DIM)))
    ).astype(np.float32)
    f = np.mod(_DIST_SCALE * inv_freq32.astype(np.float64) / (2 * np.pi), 1.0)
    f_hi = (np.round(f * 4096.0) / 4096.0).astype(np.float32)
    f_lo = (f - f_hi.astype(np.float64)).astype(np.float32)
    return f_hi.reshape(1, _HALF), f_lo.reshape(1, _HALF)


_F_HI, _F_LO = _freq_reduction_consts()


def _sc_gather(weight, idx_flat, n_tokens):
    info = plsc.get_sparse_core_info()
    nw = info.num_cores * info.num_subcores
    b_per_w = n_tokens // nw
    chunk = 64
    n_chunks = b_per_w // chunk
    mesh = plsc.VectorSubcoreMesh(core_axis_name="c", subcore_axis_name="s")

    @functools.partial(
        pl.kernel,
        mesh=mesh,
        out_type=jax.ShapeDtypeStruct((n_tokens, _DIM), jnp.float32),
        scratch_types=[
            pltpu.VMEM((b_per_w,), jnp.int32),
            pltpu.VMEM((chunk, _DIM), jnp.float32),
            pltpu.SemaphoreType.DMA,
        ],
    )
    def gather_kernel(table_hbm, idx_hbm, out_hbm, idx_v, rows_v, sem):
        wid = lax.axis_index("s") * info.num_cores + lax.axis_index("c")
        base = wid * b_per_w
        pltpu.sync_copy(idx_hbm.at[pl.ds(base, b_per_w)], idx_v)

        def body(j, carry):
            pltpu.async_copy(
                table_hbm.at[idx_v.at[pl.ds(j * chunk, chunk)]], rows_v, sem
            ).wait()
            pltpu.sync_copy(rows_v, out_hbm.at[pl.ds(base + j * chunk, chunk)])
            return carry

        lax.fori_loop(0, n_chunks, body, 0)

    return gather_kernel(weight, idx_flat)


def _horner(w, coef):
    acc = jnp.full_like(w, coef[-1])
    for c in coef[-2::-1]:
        acc = acc * w + c
    return acc


def _rotary_body(f_hi_ref, f_lo_ref, n_ref, x_ref, o_ref):
    n = n_ref[...]
    f_hi = f_hi_ref[...]
    f_lo = f_lo_ref[...]
    z = n * f_hi + n * f_lo
    v = z - jnp.floor(z + 0.5)
    w = v * v
    c = _horner(w, _COS_COEF)
    s = v * _horner(w, _SIN_COEF)
    x1 = x_ref[:, :_HALF]
    x2 = x_ref[:, _HALF:]
    o_ref[:, :_HALF] = x1 * c - x2 * s
    o_ref[:, _HALF:] = x2 * c + x1 * s


def _rotary_tc(embeds, nsub):
    n_tokens = embeds.shape[0]
    t = 512
    grid = (n_tokens // t,)
    return pl.pallas_call(
        _rotary_body,
        grid=grid,
        in_specs=[
            pl.BlockSpec((1, _HALF), lambda i: (0, 0)),
            pl.BlockSpec((1, _HALF), lambda i: (0, 0)),
            pl.BlockSpec((t, 1), lambda i: (i, 0)),
            pl.BlockSpec((t, _DIM), lambda i: (i, 0)),
        ],
        out_specs=pl.BlockSpec((t, _DIM), lambda i: (i, 0)),
        out_shape=jax.ShapeDtypeStruct((n_tokens, _DIM), jnp.float32),
    )(jnp.asarray(_F_HI), jnp.asarray(_F_LO), nsub, embeds)


def kernel(ids, ids_sub, weight):
    b, s = ids.shape
    n = b * s
    idx = ids.reshape(n)
    nsub = ids_sub.astype(jnp.float32).reshape(n, 1)
    embeds = _sc_gather(weight, idx, n)
    out = _rotary_tc(embeds, nsub)
    return out.reshape(b, s, _DIM)
